# Initial kernel scaffold; baseline (speedup 1.0000x reference)
import jax
import jax.numpy as jnp
from jax.experimental import pallas as pl


def kernel(feature, edge_index, W, b):
    raise NotImplementedError("write your pallas kernel here")


# speedup vs baseline: 8.1348x; 8.1348x over previous
import functools

import jax
import jax.numpy as jnp
from jax import lax
from jax.experimental import pallas as pl
from jax.experimental.pallas import tpu as pltpu
from jax.experimental.pallas import tpu_sc as plsc

_NODES = 10000
_EDGES = 320000
_D = 128

_NC = 2
_NS = 16
_NW = _NC * _NS
_EPW = _EDGES // _NW
_B = 80
_NCH = _EPW // _B
_NG = 5
_CPG = _NCH // _NG
_NPAD = 10240
_RPS = _NPAD // _NS
_ZR = 64


def _sc_segment_sum(src4d, dst4d, feature):
    mesh = plsc.VectorSubcoreMesh(core_axis_name="c", subcore_axis_name="s")

    @functools.partial(
        pl.kernel,
        out_type=[
            jax.ShapeDtypeStruct((_NC, _NPAD, _D), jnp.float32),
            jax.ShapeDtypeStruct((_NC, _NPAD), jnp.float32),
        ],
        mesh=mesh,
        scratch_types=[
            pltpu.VMEM((_CPG, _B), jnp.int32),
            pltpu.VMEM((_CPG, _B), jnp.int32),
            pltpu.VMEM((_B, _D), jnp.float32),
            pltpu.VMEM((_B,), jnp.float32),
            pltpu.VMEM((_ZR, _D), jnp.float32),
            pltpu.VMEM((_RPS,), jnp.float32),
            pltpu.VMEM_SHARED((_NPAD, _D), jnp.float32),
            pltpu.VMEM_SHARED((_NPAD,), jnp.float32),
            pltpu.SemaphoreType.DMA,
        ],
    )
    def sc(src_hbm, dst_hbm, feat_hbm, acc_out, deg_out,
           src_v, dst_v, rows_v, ones_v, zf_v, zd_v, acc_sh, deg_sh, sem):
        c = lax.axis_index("c")
        s = lax.axis_index("s")
        wid = s * _NC + c

        zero16 = jnp.zeros((16,), jnp.float32)
        one16 = jnp.ones((16,), jnp.float32)

        for k in range(_B // 16):
            ones_v[pl.ds(16 * k, 16)] = one16

        def init_zf(i, carry):
            for k in range(_D // 16):
                zf_v[i, pl.ds(16 * k, 16)] = zero16
            return carry

        lax.fori_loop(0, _ZR, init_zf, 0)

        def init_zd(i, carry):
            zd_v[pl.ds(16 * i, 16)] = zero16
            return carry

        lax.fori_loop(0, _RPS // 16, init_zd, 0)

        for k in range(_RPS // _ZR):
            pltpu.sync_copy(zf_v, acc_sh.at[pl.ds(s * _RPS + k * _ZR, _ZR)])
        pltpu.sync_copy(zd_v, deg_sh.at[pl.ds(s * _RPS, _RPS)])
        plsc.subcore_barrier()

        for g in range(_NG):
            pltpu.sync_copy(src_hbm.at[wid, g], src_v)
            pltpu.sync_copy(dst_hbm.at[wid, g], dst_v)

            def body(j, carry):
                sidx = src_v.at[j]
                didx = dst_v.at[j]
                pltpu.async_copy(feat_hbm.at[sidx], rows_v, sem).wait()
                pltpu.sync_copy(rows_v, acc_sh.at[didx], add=True)
                pltpu.sync_copy(ones_v, deg_sh.at[didx], add=True)
                return carry

            lax.fori_loop(0, _CPG, body, 0)

        plsc.subcore_barrier()

        pltpu.sync_copy(acc_sh.at[pl.ds(s * _RPS, _RPS)],
                        acc_out.at[c, pl.ds(s * _RPS, _RPS)])
        pltpu.sync_copy(deg_sh.at[pl.ds(s * _RPS, _RPS)],
                        deg_out.at[c, pl.ds(s * _RPS, _RPS)])

    return sc(src4d, dst4d, feature)


def _tc_finish(acc2, deg2, W, b2):
    def body(acc_ref, deg_ref, w_ref, b_ref, out_ref):
        a = acc_ref[0, :_NODES] + acc_ref[1, :_NODES]
        d = deg_ref[0, :_NODES] + deg_ref[1, :_NODES]
        d = jnp.reshape(jnp.maximum(d, 1.0), (_NODES, 1))
        h = a / d
        y = lax.dot_general(h, w_ref[...], (((1,), (1,)), ((), ())),
                            preferred_element_type=jnp.float32)
        out_ref[...] = jnp.maximum(y + b_ref[...], 0.0)

    return pl.pallas_call(
        body,
        out_shape=jax.ShapeDtypeStruct((_NODES, _D), jnp.float32),
    )(acc2, deg2, W, b2)


def kernel(feature, edge_index, W, b):
    src4d = edge_index[0].astype(jnp.int32).reshape(_NW, _NG, _CPG, _B)
    dst4d = edge_index[1].astype(jnp.int32).reshape(_NW, _NG, _CPG, _B)
    acc2, deg2 = _sc_segment_sum(src4d, dst4d, feature)
    return _tc_finish(acc2, deg2, W, b.reshape(1, _D))


# speedup vs baseline: 10.2409x; 1.2589x over previous
import functools

import jax
import jax.numpy as jnp
from jax import lax
from jax.experimental import pallas as pl
from jax.experimental.pallas import tpu as pltpu
from jax.experimental.pallas import tpu_sc as plsc

_NODES = 10000
_EDGES = 320000
_D = 128

_NC = 2
_NS = 16
_NW = _NC * _NS
_EPW = _EDGES // _NW
_B = 40
_NCH = _EPW // _B
_NG = 10
_CPG = _NCH // _NG
_NPAD = 10240
_RPS = _NPAD // _NS
_ZR = 64


def _sc_segment_sum(src4d, dst4d, feature):
    mesh = plsc.VectorSubcoreMesh(core_axis_name="c", subcore_axis_name="s")

    @functools.partial(
        pl.kernel,
        out_type=[
            jax.ShapeDtypeStruct((_NC, _NPAD, _D), jnp.float32),
            jax.ShapeDtypeStruct((_NC, _NPAD), jnp.float32),
        ],
        mesh=mesh,
        scratch_types=[
            pltpu.VMEM((2, _CPG, _B), jnp.int32),
            pltpu.VMEM((2, _CPG, _B), jnp.int32),
            pltpu.VMEM((2, _B, _D), jnp.float32),
            pltpu.VMEM((_B,), jnp.float32),
            pltpu.VMEM((_ZR, _D), jnp.float32),
            pltpu.VMEM((_RPS,), jnp.float32),
            pltpu.VMEM_SHARED((_NPAD, _D), jnp.float32),
            pltpu.VMEM_SHARED((_NPAD,), jnp.float32),
            pltpu.SemaphoreType.DMA,
            pltpu.SemaphoreType.DMA,
            pltpu.SemaphoreType.DMA,
            pltpu.SemaphoreType.DMA,
            pltpu.SemaphoreType.DMA,
            pltpu.SemaphoreType.DMA,
            pltpu.SemaphoreType.DMA,
        ],
    )
    def sc(src_hbm, dst_hbm, feat_hbm, acc_out, deg_out,
           src_v, dst_v, rows_v, ones_v, zf_v, zd_v, acc_sh, deg_sh,
           g0, g1, s0, s1, d0, d1, isem):
        c = lax.axis_index("c")
        s = lax.axis_index("s")
        wid = s * _NC + c
        gsem = (g0, g1)
        ssem = (s0, s1)
        dsem = (d0, d1)

        zero16 = jnp.zeros((16,), jnp.float32)
        one16 = jnp.ones((16,), jnp.float32)

        for k in range(_B // 16):
            ones_v[pl.ds(16 * k, 16)] = one16
        ones_v[pl.ds(_B - 16, 16)] = one16

        def init_zf(i, carry):
            for k in range(_D // 16):
                zf_v[i, pl.ds(16 * k, 16)] = zero16
            return carry

        lax.fori_loop(0, _ZR, init_zf, 0)

        def init_zd(i, carry):
            zd_v[pl.ds(16 * i, 16)] = zero16
            return carry

        lax.fori_loop(0, _RPS // 16, init_zd, 0)

        def gather_start(p, j, slot):
            pltpu.async_copy(feat_hbm.at[src_v.at[p, j]], rows_v.at[slot],
                             gsem[slot])

        def gather_wait(p, j, slot):
            pltpu.make_async_copy(feat_hbm.at[src_v.at[p, j]],
                                  rows_v.at[slot], gsem[slot]).wait()

        def stage_start(g, p):
            pltpu.async_copy(src_hbm.at[wid, g], src_v.at[p], isem)
            pltpu.async_copy(dst_hbm.at[wid, g], dst_v.at[p], isem)

        def stage_wait(g, p):
            pltpu.make_async_copy(src_hbm.at[wid, g], src_v.at[p],
                                  isem).wait()
            pltpu.make_async_copy(dst_hbm.at[wid, g], dst_v.at[p],
                                  isem).wait()

        stage_start(0, 0)
        stage_wait(0, 0)
        gather_start(0, 0, 0)
        gather_start(0, 1, 1)

        for k in range(_RPS // _ZR):
            pltpu.sync_copy(zf_v, acc_sh.at[pl.ds(s * _RPS + k * _ZR, _ZR)])
        pltpu.sync_copy(zd_v, deg_sh.at[pl.ds(s * _RPS, _RPS)])
        plsc.subcore_barrier()

        def chunk(p, j, slot):
            gather_wait(p, j, slot)
            pltpu.async_copy(rows_v.at[slot], acc_sh.at[dst_v.at[p, j]],
                             ssem[slot], add=True)
            pltpu.async_copy(ones_v, deg_sh.at[dst_v.at[p, j]],
                             dsem[slot], add=True)
            pltpu.make_async_copy(rows_v.at[slot], acc_sh.at[dst_v.at[p, j]],
                                  ssem[slot]).wait()
            pltpu.make_async_copy(ones_v, deg_sh.at[dst_v.at[p, j]],
                                  dsem[slot]).wait()

        def group(g, carry):
            p = g % 2

            @pl.when(g + 1 < _NG)
            def _():
                stage_start(g + 1, 1 - p)

            def body(i, carry2):
                for b in range(2):
                    j = 2 * i + b
                    chunk(p, j, b)

                    @pl.when(j + 2 < _CPG)
                    def _():
                        gather_start(p, j + 2, b)

                return carry2

            lax.fori_loop(0, _CPG // 2, body, 0)
            chunk(p, _CPG - 1, (_CPG - 1) % 2)

            @pl.when(g + 1 < _NG)
            def _():
                stage_wait(g + 1, 1 - p)
                gather_start(1 - p, 0, 0)
                gather_start(1 - p, 1, 1)

            return carry

        lax.fori_loop(0, _NG, group, 0)
        plsc.subcore_barrier()

        pltpu.sync_copy(acc_sh.at[pl.ds(s * _RPS, _RPS)],
                        acc_out.at[c, pl.ds(s * _RPS, _RPS)])
        pltpu.sync_copy(deg_sh.at[pl.ds(s * _RPS, _RPS)],
                        deg_out.at[c, pl.ds(s * _RPS, _RPS)])

    return sc(src4d, dst4d, feature)


def _tc_finish(acc2, deg2, W, b2):
    def body(acc_ref, deg_ref, w_ref, b_ref, out_ref):
        a = acc_ref[0, :_NODES] + acc_ref[1, :_NODES]
        d = deg_ref[0, :_NODES] + deg_ref[1, :_NODES]
        d = jnp.reshape(jnp.maximum(d, 1.0), (_NODES, 1))
        h = a / d
        y = lax.dot_general(h, w_ref[...], (((1,), (1,)), ((), ())),
                            preferred_element_type=jnp.float32)
        out_ref[...] = jnp.maximum(y + b_ref[...], 0.0)

    return pl.pallas_call(
        body,
        out_shape=jax.ShapeDtypeStruct((_NODES, _D), jnp.float32),
    )(acc2, deg2, W, b2)


def kernel(feature, edge_index, W, b):
    src4d = edge_index[0].astype(jnp.int32).reshape(_NW, _NG, _CPG, _B)
    dst4d = edge_index[1].astype(jnp.int32).reshape(_NW, _NG, _CPG, _B)
    acc2, deg2 = _sc_segment_sum(src4d, dst4d, feature)
    return _tc_finish(acc2, deg2, W, b.reshape(1, _D))
